# baseline (speedup 1.0000x reference)
import functools

import jax
import jax.numpy as jnp
from jax import lax
from jax.experimental import pallas as pl
from jax.experimental.pallas import tpu as pltpu
from jax.experimental.pallas import tpu_sc as plsc


def _vq_idx_body(z_ref, cb_ref, idx_ref, cbn_ref):
    @pl.when(jnp.logical_and(pl.program_id(0) == 0, pl.program_id(1) == 0))
    def _():
        cb0 = cb_ref[...]
        cbn_ref[...] = 0.5 * jnp.sum(cb0 * cb0, axis=1, keepdims=True)

    zb = z_ref[0]
    cb = cb_ref[...]
    scores = jax.lax.dot_general(
        cb, zb, (((1,), (0,)), ((), ())),
        preferred_element_type=jnp.float32)
    rank = scores - cbn_ref[...]
    idx = jnp.argmax(rank, axis=0).astype(jnp.int32)
    idx_ref[0, 0, 0] = idx


def _tc_indices(z3, codebook, T):
    B, D, HW = z3.shape
    K = codebook.shape[0]
    NT = HW // T
    idx = pl.pallas_call(
        _vq_idx_body,
        grid=(B, NT),
        in_specs=[
            pl.BlockSpec((1, D, T), lambda b, t: (b, 0, t)),
            pl.BlockSpec((K, D), lambda b, t: (0, 0)),
        ],
        out_specs=pl.BlockSpec((1, 1, 1, T), lambda b, t: (b, t, 0, 0)),
        out_shape=jax.ShapeDtypeStruct((B, NT, 1, T), jnp.int32),
        scratch_shapes=[pltpu.VMEM((K, 1), jnp.float32)],
    )(z3, codebook)
    return idx.reshape(B, HW)


def _make_sc_gather(N, D, C):
    info = plsc.get_sparse_core_info()
    NC, NS = info.num_cores, info.num_subcores
    NW = NC * NS
    n_per_w = N // NW
    nch = n_per_w // C
    mesh = plsc.VectorSubcoreMesh(core_axis_name="c", subcore_axis_name="s")

    @functools.partial(
        pl.kernel, mesh=mesh,
        out_type=jax.ShapeDtypeStruct((N, D), jnp.float32),
        scratch_types=[
            pltpu.VMEM((C,), jnp.int32),
            pltpu.VMEM((C, D), jnp.float32),
            pltpu.SemaphoreType.DMA,
        ],
    )
    def gather(table_hbm, idx_hbm, out_hbm, idx_v, rows_v, sem):
        wid = lax.axis_index("s") * NC + lax.axis_index("c")
        base = wid * n_per_w

        def body(i, carry):
            off = base + i * C
            pltpu.sync_copy(idx_hbm.at[pl.ds(off, C)], idx_v)
            pltpu.async_copy(table_hbm.at[idx_v], rows_v, sem).wait()
            pltpu.sync_copy(rows_v, out_hbm.at[pl.ds(off, C)])
            return carry

        lax.fori_loop(0, nch, body, 0)

    return gather


def kernel(z, codebook):
    B, D, H, W = z.shape
    HW = H * W
    z3 = z.reshape(B, D, HW)
    T = min(4096, HW)
    idx = _tc_indices(z3, codebook, T)
    N = B * HW
    x = _make_sc_gather(N, D, 128)(codebook, idx.reshape(N))
    return x.reshape(B, HW, D), idx
